# Initial kernel scaffold; baseline (speedup 1.0000x reference)
import functools

import jax
import jax.numpy as jnp
from jax import lax
from jax.experimental import pallas as pl
from jax.experimental.pallas import tpu as pltpu
from jax.experimental.pallas import tpu_sc as plsc

N = 10000
E = 320000
D = 128

NC = 2
NS = 16
NW = NC * NS
EPW = E // NW
K = 80
NCHUNK = EPW // K
RPT = N // NS
ZROWS = 125


def _sc_aggregate(x, adj_indices, adj_values):
    mesh = plsc.VectorSubcoreMesh(core_axis_name="c", subcore_axis_name="s")

    @functools.partial(
        pl.kernel,
        out_type=jax.ShapeDtypeStruct((NC, N, D), jnp.float32),
        mesh=mesh,
        scratch_types=[
            pltpu.VMEM((K,), jnp.int32),
            pltpu.VMEM((K,), jnp.int32),
            pltpu.VMEM((K,), jnp.float32),
            pltpu.VMEM((K, D), jnp.float32),
            pltpu.VMEM((ZROWS, D), jnp.float32),
            pltpu.VMEM_SHARED((N, D), jnp.float32),
            pltpu.SemaphoreType.DMA,
        ],
    )
    def agg_kernel(x_hbm, adj_hbm, val_hbm, out_hbm,
                   src_v, dst_v, val_v, rows_v, zero_v, acc, sem):
        c = lax.axis_index("c")
        s = lax.axis_index("s")

        zeros16 = jnp.zeros((16,), jnp.float32)

        def zero_row(r, _):
            for j in range(D // 16):
                zero_v[r, pl.ds(j * 16, 16)] = zeros16
            return 0

        lax.fori_loop(0, ZROWS, zero_row, 0)
        for rep in range(RPT // ZROWS):
            pltpu.sync_copy(
                zero_v, acc.at[pl.ds(s * RPT + rep * ZROWS, ZROWS)])
        plsc.subcore_barrier()

        worker_base = (c * NS + s) * EPW

        def chunk_body(g, _):
            base = worker_base + g * K
            pltpu.sync_copy(adj_hbm.at[1, pl.ds(base, K)], src_v)
            pltpu.sync_copy(adj_hbm.at[0, pl.ds(base, K)], dst_v)
            pltpu.sync_copy(val_hbm.at[pl.ds(base, K)], val_v)
            pltpu.async_copy(x_hbm.at[src_v], rows_v, sem).wait()

            def scale_row(e, _):
                v = val_v[e]
                for j in range(D // 16):
                    sl = pl.ds(j * 16, 16)
                    rows_v[e, sl] = rows_v[e, sl] * v
                return 0

            lax.fori_loop(0, K, scale_row, 0)

            pltpu.sync_copy(rows_v, acc.at[dst_v], add=True)
            return 0

        lax.fori_loop(0, NCHUNK, chunk_body, 0)
        plsc.subcore_barrier()

        pltpu.sync_copy(acc.at[pl.ds(s * RPT, RPT)],
                        out_hbm.at[c, pl.ds(s * RPT, RPT)])

    return agg_kernel(x, adj_indices, adj_values)


def _tc_combine(partials, W):
    BM = 1000

    def combine_kernel(p_ref, w_ref, out_ref):
        p = p_ref[0] + p_ref[1]
        out_ref[...] = lax.dot_general(
            p, w_ref[...], (((1,), (1,)), ((), ())),
            preferred_element_type=jnp.float32,
            precision=lax.Precision.HIGHEST)

    return pl.pallas_call(
        combine_kernel,
        grid=(N // BM,),
        in_specs=[
            pl.BlockSpec((NC, BM, D), lambda i: (0, i, 0)),
            pl.BlockSpec((D, D), lambda i: (0, 0)),
        ],
        out_specs=pl.BlockSpec((BM, D), lambda i: (i, 0)),
        out_shape=jax.ShapeDtypeStruct((N, D), jnp.float32),
    )(partials, W)


def kernel(x, adj_indices, adj_values, W):
    partials = _sc_aggregate(x, adj_indices, adj_values)
    return _tc_combine(partials, W)


# speedup vs baseline: 4.5009x; 4.5009x over previous
import functools

import jax
import jax.numpy as jnp
from jax import lax
from jax.experimental import pallas as pl
from jax.experimental.pallas import tpu as pltpu
from jax.experimental.pallas import tpu_sc as plsc

N = 10000
E = 320000
D = 128

NC = 2
NS = 16
NW = NC * NS
EPW = E // NW
K = 80
NCHUNK = EPW // K
RPT = 624
TAIL = N - NS * RPT
ZROWS = 208


def _sc_aggregate(x, dst_idx, src_idx, adj_values):
    mesh = plsc.VectorSubcoreMesh(core_axis_name="c", subcore_axis_name="s")

    @functools.partial(
        pl.kernel,
        out_type=jax.ShapeDtypeStruct((NC, N, D), jnp.float32),
        mesh=mesh,
        scratch_types=[
            pltpu.VMEM((K,), jnp.int32),
            pltpu.VMEM((K,), jnp.int32),
            pltpu.VMEM((K,), jnp.float32),
            pltpu.VMEM((K, D), jnp.float32),
            pltpu.VMEM((ZROWS, D), jnp.float32),
            pltpu.VMEM_SHARED((N, D), jnp.float32),
            pltpu.SemaphoreType.DMA,
        ],
    )
    def agg_kernel(x_hbm, dst_hbm, src_hbm, val_hbm, out_hbm,
                   src_v, dst_v, val_v, rows_v, zero_v, acc, sem):
        c = lax.axis_index("c")
        s = lax.axis_index("s")

        zeros16 = jnp.zeros((16,), jnp.float32)

        def zero_row(r, _):
            for j in range(D // 16):
                zero_v[r, pl.ds(j * 16, 16)] = zeros16
            return 0

        lax.fori_loop(0, ZROWS, zero_row, 0)
        for rep in range(RPT // ZROWS):
            pltpu.sync_copy(
                zero_v, acc.at[pl.ds(s * RPT + rep * ZROWS, ZROWS)])

        @pl.when(s == NS - 1)
        def _():
            pltpu.sync_copy(zero_v.at[pl.ds(0, TAIL)],
                            acc.at[pl.ds(NS * RPT, TAIL)])

        plsc.subcore_barrier()

        worker_base = (c * NS + s) * EPW

        def chunk_body(g, _):
            base = worker_base + g * K
            pltpu.sync_copy(src_hbm.at[pl.ds(base, K)], src_v)
            pltpu.sync_copy(dst_hbm.at[pl.ds(base, K)], dst_v)
            pltpu.sync_copy(val_hbm.at[pl.ds(base, K)], val_v)
            pltpu.async_copy(x_hbm.at[src_v], rows_v, sem).wait()

            def scale_group(gi, _):
                vv = val_v[pl.ds(gi * 16, 16)]
                for i in range(16):
                    e = gi * 16 + i
                    vs = vv[i]
                    for j in range(D // 16):
                        sl = pl.ds(j * 16, 16)
                        rows_v[e, sl] = rows_v[e, sl] * vs
                return 0

            lax.fori_loop(0, K // 16, scale_group, 0)

            pltpu.sync_copy(rows_v, acc.at[dst_v], add=True)
            return 0

        lax.fori_loop(0, NCHUNK, chunk_body, 0)
        plsc.subcore_barrier()

        pltpu.sync_copy(acc.at[pl.ds(s * RPT, RPT)],
                        out_hbm.at[c, pl.ds(s * RPT, RPT)])

        @pl.when(s == NS - 1)
        def _():
            pltpu.sync_copy(acc.at[pl.ds(NS * RPT, TAIL)],
                            out_hbm.at[c, pl.ds(NS * RPT, TAIL)])

    return agg_kernel(x, dst_idx, src_idx, adj_values)


def _tc_combine(partials, W):
    BM = 1000

    def combine_kernel(p_ref, w_ref, out_ref):
        p = p_ref[0] + p_ref[1]
        out_ref[...] = lax.dot_general(
            p, w_ref[...], (((1,), (1,)), ((), ())),
            preferred_element_type=jnp.float32,
            precision=lax.Precision.HIGHEST)

    return pl.pallas_call(
        combine_kernel,
        grid=(N // BM,),
        in_specs=[
            pl.BlockSpec((NC, BM, D), lambda i: (0, i, 0)),
            pl.BlockSpec((D, D), lambda i: (0, 0)),
        ],
        out_specs=pl.BlockSpec((BM, D), lambda i: (i, 0)),
        out_shape=jax.ShapeDtypeStruct((N, D), jnp.float32),
    )(partials, W)


def kernel(x, adj_indices, adj_values, W):
    dst_idx = adj_indices[0]
    src_idx = adj_indices[1]
    partials = _sc_aggregate(x, dst_idx, src_idx, adj_values)
    return _tc_combine(partials, W)
